# baseline (speedup 1.0000x reference)
import jax
import jax.numpy as jnp
from jax.experimental import pallas as pl
from jax.experimental.pallas import tpu as pltpu

_INV_SQRT2 = 0.7071067811865476
_D = tuple(c * _INV_SQRT2 ** (2 * k + 1) for k, c in enumerate((
    1.1140025122481443, -0.3305722968551459, 0.07111085796400862,
    -0.008781295218724158, 0.0005607269702987659, -1.4290652414927774e-05)))

_TM = 16384


def _erf_gelu(y):
    u = y * y
    q = _D[5]
    for c in (_D[4], _D[3], _D[2], _D[1], _D[0]):
        q = q * u + c
    erf = jnp.clip(y * q, -1.0, 1.0)
    return 0.5 * y * (1.0 + erf)


def _body(x0_ref, x1_ref, x2_ref, w_ref, b_ref, o_ref):
    w = w_ref[...]
    f = w.shape[1]
    tm = o_ref.shape[0]
    q = tm // 4
    c2 = jnp.dot(x2_ref[...], w[2 * f:3 * f, :],
                 preferred_element_type=jnp.float32) + b_ref[...]
    w1 = w[f:2 * f, :]
    z1e = jnp.dot(x1_ref[pl.Slice(0, tm // 4, 2), :],
                  w1, preferred_element_type=jnp.float32) + c2
    z1o = jnp.dot(x1_ref[pl.Slice(1, tm // 4, 2), :],
                  w1, preferred_element_type=jnp.float32) + c2
    w0 = w[0:f, :]
    for p, z in ((0, z1e), (1, z1e), (2, z1o), (3, z1o)):
        x0p = x0_ref[pl.Slice(p, q, 4), :]
        y = jnp.dot(x0p, w0, preferred_element_type=jnp.float32) + z
        o_ref[pl.Slice(p, q, 4), :] = _erf_gelu(y)


def kernel(x0, x1, x2, w, b):
    batch, t, f = x0.shape
    rows = batch * t
    xs = [x0.reshape(rows, f),
          x1[:, :t >> 1, :].reshape(rows >> 1, f),
          x2[:, :t >> 2, :].reshape(rows >> 2, f)]

    tm = _TM
    grid = (rows // tm,)

    out = pl.pallas_call(
        _body,
        out_shape=jax.ShapeDtypeStruct((rows, f), x0.dtype),
        grid=grid,
        in_specs=[
            pl.BlockSpec((tm, f), lambda i: (i, 0)),
            pl.BlockSpec((tm >> 1, f), lambda i: (i, 0)),
            pl.BlockSpec((tm >> 2, f), lambda i: (i, 0)),
            pl.BlockSpec((3 * f, f), lambda i: (0, 0)),
            pl.BlockSpec((1, f), lambda i: (0, 0)),
        ],
        out_specs=pl.BlockSpec((tm, f), lambda i: (i, 0)),
        compiler_params=pltpu.CompilerParams(
            dimension_semantics=("arbitrary",),
            vmem_limit_bytes=56 * 1024 * 1024),
    )(*xs, w, b)
    return out.reshape(batch, t, f)


# speedup vs baseline: 1.0525x; 1.0525x over previous
import jax
import jax.numpy as jnp
from jax.experimental import pallas as pl
from jax.experimental.pallas import tpu as pltpu

_INV_SQRT2 = 0.7071067811865476
_D = tuple(c * _INV_SQRT2 ** (2 * k + 1) for k, c in enumerate((
    1.1140025122481443, -0.3305722968551459, 0.07111085796400862,
    -0.008781295218724158, 0.0005607269702987659, -1.4290652414927774e-05)))

_TM = 8192


def _erf_gelu(y):
    u = y * y
    q = _D[5]
    for c in (_D[4], _D[3], _D[2], _D[1], _D[0]):
        q = q * u + c
    erf = jnp.clip(y * q, -1.0, 1.0)
    return 0.5 * y * (1.0 + erf)


def _body(x0_ref, x1_ref, x2_ref, w_ref, b_ref, o_ref):
    w = w_ref[...]
    f = w.shape[1]
    tm = o_ref.shape[0]
    q = tm // 4
    c2 = jnp.dot(x2_ref[...], w[2 * f:3 * f, :],
                 preferred_element_type=jnp.float32) + b_ref[...]
    w1 = w[f:2 * f, :]
    z1e = jnp.dot(x1_ref[pl.Slice(0, tm // 4, 2), :],
                  w1, preferred_element_type=jnp.float32) + c2
    z1o = jnp.dot(x1_ref[pl.Slice(1, tm // 4, 2), :],
                  w1, preferred_element_type=jnp.float32) + c2
    w0 = w[0:f, :]
    for p, z in ((0, z1e), (1, z1e), (2, z1o), (3, z1o)):
        x0p = x0_ref[pl.Slice(p, q, 4), :]
        y = jnp.dot(x0p, w0, preferred_element_type=jnp.float32) + z
        o_ref[pl.Slice(p, q, 4), :] = _erf_gelu(y)


def kernel(x0, x1, x2, w, b):
    batch, t, f = x0.shape
    rows = batch * t
    xs = [x0.reshape(rows, f),
          x1[:, :t >> 1, :].reshape(rows >> 1, f),
          x2[:, :t >> 2, :].reshape(rows >> 2, f)]

    tm = _TM
    grid = (rows // tm,)

    out = pl.pallas_call(
        _body,
        out_shape=jax.ShapeDtypeStruct((rows, f), x0.dtype),
        grid=grid,
        in_specs=[
            pl.BlockSpec((tm, f), lambda i: (i, 0)),
            pl.BlockSpec((tm >> 1, f), lambda i: (i, 0)),
            pl.BlockSpec((tm >> 2, f), lambda i: (i, 0)),
            pl.BlockSpec((3 * f, f), lambda i: (0, 0)),
            pl.BlockSpec((1, f), lambda i: (0, 0)),
        ],
        out_specs=pl.BlockSpec((tm, f), lambda i: (i, 0)),
        compiler_params=pltpu.CompilerParams(
            dimension_semantics=("arbitrary",),
            vmem_limit_bytes=56 * 1024 * 1024),
    )(*xs, w, b)
    return out.reshape(batch, t, f)


# speedup vs baseline: 1.0928x; 1.0382x over previous
import jax
import jax.numpy as jnp
from jax.experimental import pallas as pl
from jax.experimental.pallas import tpu as pltpu

_INV_SQRT2 = 0.7071067811865476
_D = tuple(c * _INV_SQRT2 ** (2 * k + 1) for k, c in enumerate((
    1.1140025122481443, -0.3305722968551459, 0.07111085796400862,
    -0.008781295218724158, 0.0005607269702987659, -1.4290652414927774e-05)))

_CH = 2048
_DEPTH = 3
_NOUT = 2


def _erf_gelu(y):
    u = y * y
    q = _D[5]
    for c in (_D[4], _D[3], _D[2], _D[1], _D[0]):
        q = q * u + c
    erf = jnp.clip(y * q, -1.0, 1.0)
    return 0.5 * y * (1.0 + erf)


def _compute_chunk(x0b, x1b, x2b, w, bias, ob, slot, oslot):
    f = w.shape[1]
    q = _CH // 4
    c2 = jnp.dot(x2b[slot], w[2 * f:3 * f, :],
                 preferred_element_type=jnp.float32) + bias
    w1 = w[f:2 * f, :]
    z1e = jnp.dot(x1b[slot, pl.Slice(0, q, 2), :], w1,
                  preferred_element_type=jnp.float32) + c2
    z1o = jnp.dot(x1b[slot, pl.Slice(1, q, 2), :], w1,
                  preferred_element_type=jnp.float32) + c2
    w0 = w[0:f, :]
    for p, z in ((0, z1e), (1, z1e), (2, z1o), (3, z1o)):
        y = jnp.dot(x0b[slot, pl.Slice(p, q, 4), :], w0,
                    preferred_element_type=jnp.float32) + z
        ob[oslot, pl.Slice(p, q, 4), :] = _erf_gelu(y)


def _body(x0_hbm, x1_hbm, x2_hbm, w_ref, b_ref, o_hbm,
          x0b, x1b, x2b, ob, insem, outsem):
    ch = _CH
    n_ch = x0_hbm.shape[0] // ch

    def in_copies(c, slot):
        return (
            pltpu.make_async_copy(x0_hbm.at[pl.ds(c * ch, ch), :],
                                  x0b.at[slot], insem.at[0, slot]),
            pltpu.make_async_copy(x1_hbm.at[pl.ds(c * (ch // 2), ch // 2), :],
                                  x1b.at[slot], insem.at[1, slot]),
            pltpu.make_async_copy(x2_hbm.at[pl.ds(c * (ch // 4), ch // 4), :],
                                  x2b.at[slot], insem.at[2, slot]),
        )

    def out_copy(c, oslot):
        return pltpu.make_async_copy(
            ob.at[oslot], o_hbm.at[pl.ds(c * ch, ch), :], outsem.at[oslot])

    for c in range(min(_DEPTH, n_ch)):
        for cp in in_copies(c, c):
            cp.start()

    w = w_ref[...]
    bias = b_ref[...]

    def step(c, carry):
        slot = jax.lax.rem(c, _DEPTH)
        oslot = jax.lax.rem(c, _NOUT)
        for cp in in_copies(c, slot):
            cp.wait()
        @pl.when(c >= _NOUT)
        def _():
            out_copy(c - _NOUT, oslot).wait()
        _compute_chunk(x0b, x1b, x2b, w, bias, ob, slot, oslot)
        out_copy(c, oslot).start()
        @pl.when(c + _DEPTH < n_ch)
        def _():
            for cp in in_copies(c + _DEPTH, slot):
                cp.start()
        return carry

    jax.lax.fori_loop(0, n_ch, step, 0)
    for k in range(max(0, n_ch - _NOUT), n_ch):
        out_copy(k, k % _NOUT).wait()


def kernel(x0, x1, x2, w, b):
    batch, t, f = x0.shape
    rows = batch * t
    xs = [x0.reshape(rows, f),
          x1[:, :t >> 1, :].reshape(rows >> 1, f),
          x2[:, :t >> 2, :].reshape(rows >> 2, f)]

    ch = _CH
    out = pl.pallas_call(
        _body,
        out_shape=jax.ShapeDtypeStruct((rows, f), x0.dtype),
        grid=(1,),
        in_specs=[
            pl.BlockSpec(memory_space=pltpu.MemorySpace.HBM),
            pl.BlockSpec(memory_space=pltpu.MemorySpace.HBM),
            pl.BlockSpec(memory_space=pltpu.MemorySpace.HBM),
            pl.BlockSpec((3 * f, f), lambda i: (0, 0)),
            pl.BlockSpec((1, f), lambda i: (0, 0)),
        ],
        out_specs=pl.BlockSpec(memory_space=pltpu.MemorySpace.HBM),
        scratch_shapes=[
            pltpu.VMEM((_DEPTH, ch, f), jnp.float32),
            pltpu.VMEM((_DEPTH, ch // 2, f), jnp.float32),
            pltpu.VMEM((_DEPTH, ch // 4, f), jnp.float32),
            pltpu.VMEM((_NOUT, ch, f), jnp.float32),
            pltpu.SemaphoreType.DMA((3, _DEPTH)),
            pltpu.SemaphoreType.DMA((_NOUT,)),
        ],
        compiler_params=pltpu.CompilerParams(
            dimension_semantics=("arbitrary",),
            vmem_limit_bytes=56 * 1024 * 1024),
    )(*xs, w, b)
    return out.reshape(batch, t, f)
